# baseline (speedup 1.0000x reference)
import functools

import jax
import jax.numpy as jnp
from jax import lax
from jax.experimental import pallas as pl
from jax.experimental.pallas import tpu as pltpu
from jax.experimental.pallas import tpu_sc as plsc

NUM_TABLES = 26
TABLE_ROWS = 100000
DIM = 32
BATCH = 16384

FLAT = BATCH * NUM_TABLES
CHUNK = 128
NUM_CHUNKS = FLAT // CHUNK
L = 16


def _make_kernel(num_workers):
    rows_per_w = NUM_CHUNKS // num_workers
    mesh = plsc.VectorSubcoreMesh(core_axis_name="c", subcore_axis_name="s")

    @functools.partial(
        pl.kernel,
        mesh=mesh,
        out_type=jax.ShapeDtypeStruct((FLAT, DIM), jnp.float32),
        scratch_types=[
            pltpu.VMEM((rows_per_w, CHUNK), jnp.int32),
            pltpu.VMEM((CHUNK, DIM), jnp.float32),
            pltpu.SemaphoreType.DMA,
        ],
        compiler_params=pltpu.CompilerParams(use_tc_tiling_on_sc=False),
    )
    def gather_kernel(table_hbm, idx_hbm, out_hbm, idx_v, rows_v, sem):
        wid = lax.axis_index("s") * 2 + lax.axis_index("c")
        base_row = wid * rows_per_w

        pltpu.sync_copy(idx_hbm.at[pl.ds(base_row, rows_per_w)], idx_v)

        lane = lax.iota(jnp.int32, L)
        slices_per_row = CHUNK // L

        def add_offsets(t, _):
            r = t // slices_per_row
            s = t % slices_per_row
            pos = (base_row + r) * CHUNK + s * L + lane
            offs = (pos % NUM_TABLES) * TABLE_ROWS
            idx_v[r, pl.ds(s * L, L)] = idx_v[r, pl.ds(s * L, L)] + offs
            return 0

        lax.fori_loop(0, rows_per_w * slices_per_row, add_offsets, 0)

        def do_chunk(j, _):
            pltpu.async_copy(table_hbm.at[idx_v.at[j]], rows_v, sem).wait()
            pltpu.sync_copy(rows_v, out_hbm.at[pl.ds((base_row + j) * CHUNK, CHUNK)])
            return 0

        lax.fori_loop(0, rows_per_w, do_chunk, 0)

    return gather_kernel


_kernel_32 = _make_kernel(32)


@jax.jit
def kernel(indices, embedding_table):
    idx = indices.astype(jnp.int32).reshape(NUM_CHUNKS, CHUNK)
    out = _kernel_32(embedding_table, idx)
    return out.reshape(BATCH, NUM_TABLES, DIM)


# speedup vs baseline: 1.0322x; 1.0322x over previous
import functools

import jax
import jax.numpy as jnp
from jax import lax
from jax.experimental import pallas as pl
from jax.experimental.pallas import tpu as pltpu
from jax.experimental.pallas import tpu_sc as plsc

NUM_TABLES = 26
TABLE_ROWS = 100000
DIM = 32
BATCH = 16384

FLAT = BATCH * NUM_TABLES
CHUNK = 128
NUM_CHUNKS = FLAT // CHUNK
L = 16


def _make_kernel(num_workers):
    rows_per_w = NUM_CHUNKS // num_workers
    mesh = plsc.VectorSubcoreMesh(core_axis_name="c", subcore_axis_name="s")

    @functools.partial(
        pl.kernel,
        mesh=mesh,
        out_type=jax.ShapeDtypeStruct((FLAT, DIM), jnp.float32),
        scratch_types=[
            pltpu.VMEM((rows_per_w, CHUNK), jnp.int32),
            pltpu.VMEM((2, CHUNK, DIM), jnp.float32),
            pltpu.SemaphoreType.DMA((2,)),
            pltpu.SemaphoreType.DMA((2,)),
        ],
        compiler_params=pltpu.CompilerParams(use_tc_tiling_on_sc=False),
    )
    def gather_kernel(table_hbm, idx_hbm, out_hbm, idx_v, rows_v, gsem, ssem):
        wid = lax.axis_index("s") * 2 + lax.axis_index("c")
        base_row = wid * rows_per_w

        pltpu.sync_copy(idx_hbm.at[pl.ds(base_row, rows_per_w)], idx_v)

        lane = lax.iota(jnp.int32, L)

        def add_offsets(r, _):
            row_base = (base_row + r) * CHUNK
            for s in range(CHUNK // L):
                pos = row_base + s * L + lane
                offs = (pos % NUM_TABLES) * TABLE_ROWS
                idx_v[r, pl.ds(s * L, L)] = idx_v[r, pl.ds(s * L, L)] + offs
            return 0

        lax.fori_loop(0, rows_per_w, add_offsets, 0)

        def fire_gather(j, buf):
            pltpu.async_copy(table_hbm.at[idx_v.at[j]], rows_v.at[buf],
                             gsem.at[buf])

        def wait_gather(j, buf):
            pltpu.make_async_copy(table_hbm.at[idx_v.at[j]], rows_v.at[buf],
                                  gsem.at[buf]).wait()

        def out_slab(j):
            return out_hbm.at[pl.ds((base_row + j) * CHUNK, CHUNK)]

        def wait_store(buf):
            pltpu.make_async_copy(rows_v.at[buf], out_slab(0),
                                  ssem.at[buf]).wait()

        fire_gather(0, 0)

        def step(p, _):
            for buf in (0, 1):
                j = 2 * p + buf

                @pl.when(j >= 1)
                def _():
                    wait_store(1 - buf)

                @pl.when(j + 1 < rows_per_w)
                def _():
                    fire_gather(j + 1, 1 - buf)

                wait_gather(j, buf)
                pltpu.async_copy(rows_v.at[buf], out_slab(j), ssem.at[buf])
            return 0

        lax.fori_loop(0, rows_per_w // 2, step, 0)
        wait_store(1)

    return gather_kernel


_kernel_32 = _make_kernel(32)


@jax.jit
def kernel(indices, embedding_table):
    idx_flat = lax.optimization_barrier(indices.astype(jnp.int32).reshape(-1))
    idx = idx_flat.reshape(NUM_CHUNKS, CHUNK)
    tab_flat = lax.optimization_barrier(embedding_table.reshape(-1))
    tab = tab_flat.reshape(TABLE_ROWS * NUM_TABLES, DIM)
    out = _kernel_32(tab, idx)
    return out.reshape(BATCH, NUM_TABLES, DIM)
